# Initial kernel scaffold; baseline (speedup 1.0000x reference)
import functools

import jax
import jax.numpy as jnp
from jax import lax
from jax.experimental import pallas as pl
from jax.experimental.pallas import tpu as pltpu
from jax.experimental.pallas import tpu_sc as plsc

N = 10000
K = 17
D = 128
E = N * (K - 1)
EPS = 1e-5

NC, NS = 2, 16
NW = NC * NS
CH = 128
NCHUNK = E // CH
CPW = -(-NCHUNK // NW)
RPT = N // NS

PT = 5328
PP = PT * NW
DLOOP = PT // 16

_MESH = plsc.VectorSubcoreMesh(
    core_axis_name="c", subcore_axis_name="s", num_cores=NC, num_subcores=NS)


@functools.partial(
    pl.kernel,
    out_type=jax.ShapeDtypeStruct((E, D), jnp.float32),
    mesh=_MESH,
    scratch_types=[
        pltpu.VMEM((CH,), jnp.int32),
        pltpu.VMEM((CH, D), jnp.float32),
        pltpu.SemaphoreType.DMA,
    ],
)
def _sc_gather(table_hbm, src_hbm, out_hbm, idx_v, rows_v, sem):
    wid = lax.axis_index("s") * NC + lax.axis_index("c")
    for i in range(CPW):
        cid = wid + i * NW

        @pl.when(cid < NCHUNK)
        def _():
            base = cid * CH
            pltpu.sync_copy(src_hbm.at[pl.ds(base, CH)], idx_v)
            pltpu.async_copy(table_hbm.at[idx_v], rows_v, sem).wait()
            pltpu.sync_copy(rows_v, out_hbm.at[pl.ds(base, CH)])


@functools.partial(
    pl.kernel,
    out_type=(
        jax.ShapeDtypeStruct((NC, N, D), jnp.float32),
        jax.ShapeDtypeStruct((NC, N, 16), jnp.float32),
    ),
    mesh=_MESH,
    scratch_types=[
        pltpu.VMEM((CH,), jnp.int32),
        pltpu.VMEM((CH, D), jnp.float32),
        pltpu.VMEM((CH, 16), jnp.float32),
        pltpu.VMEM_SHARED((N, D), jnp.float32),
        pltpu.VMEM_SHARED((N, 16), jnp.float32),
    ],
)
def _sc_scatter(v_hbm, src_hbm, znd_hbm, z16_hbm, ones_hbm,
                s_out, cnt_out, idx_v, rows_v, ones_v, s_sh, cnt_sh):
    c = lax.axis_index("c")
    s = lax.axis_index("s")
    wid = s * NC + c
    row0 = s * RPT
    pltpu.sync_copy(znd_hbm.at[pl.ds(row0, RPT)], s_sh.at[pl.ds(row0, RPT)])
    pltpu.sync_copy(z16_hbm.at[pl.ds(row0, RPT)], cnt_sh.at[pl.ds(row0, RPT)])
    pltpu.sync_copy(ones_hbm, ones_v)
    plsc.subcore_barrier()
    for i in range(CPW):
        cid = wid + i * NW

        @pl.when(cid < NCHUNK)
        def _():
            base = cid * CH
            pltpu.sync_copy(src_hbm.at[pl.ds(base, CH)], idx_v)
            pltpu.sync_copy(v_hbm.at[pl.ds(base, CH)], rows_v)
            pltpu.sync_copy(rows_v, s_sh.at[idx_v], add=True)
            pltpu.sync_copy(ones_v, cnt_sh.at[idx_v], add=True)
    plsc.subcore_barrier()
    pltpu.sync_copy(s_sh.at[pl.ds(row0, RPT)], s_out.at[c, pl.ds(row0, RPT)])
    pltpu.sync_copy(cnt_sh.at[pl.ds(row0, RPT)],
                    cnt_out.at[c, pl.ds(row0, RPT)])


@functools.partial(
    pl.kernel,
    out_type=jax.ShapeDtypeStruct((PP,), jnp.float32),
    mesh=_MESH,
    scratch_types=[
        pltpu.VMEM((N,), jnp.float32),
        pltpu.VMEM((N,), jnp.float32),
        pltpu.VMEM((PT,), jnp.int32),
        pltpu.VMEM((PT,), jnp.int32),
        pltpu.VMEM((PT,), jnp.float32),
    ],
)
def _sc_dist(h0_hbm, h1_hbm, srcf_hbm, dstf_hbm, out_hbm,
             c0, c1, si, di, ob):
    wid = lax.axis_index("s") * NC + lax.axis_index("c")
    base = wid * PT
    pltpu.sync_copy(h0_hbm, c0)
    pltpu.sync_copy(h1_hbm, c1)
    pltpu.sync_copy(srcf_hbm.at[pl.ds(base, PT)], si)
    pltpu.sync_copy(dstf_hbm.at[pl.ds(base, PT)], di)

    def body(j, carry):
        sv = si[pl.ds(j * 16, 16)]
        dv = di[pl.ds(j * 16, 16)]
        a0 = plsc.load_gather(c0, [sv])
        a1 = plsc.load_gather(c1, [sv])
        b0 = plsc.load_gather(c0, [dv])
        b1 = plsc.load_gather(c1, [dv])
        d0 = b0 - a0
        d1 = b1 - a1
        ob[pl.ds(j * 16, 16)] = d0 * d0 + d1 * d1
        return carry

    lax.fori_loop(0, DLOOP, body, 0)
    pltpu.sync_copy(ob, out_hbm.at[pl.ds(base, PT)])


def _lin_body(h_ref, w_ref, b_ref, o_ref):
    o_ref[...] = jnp.dot(h_ref[...], w_ref[...],
                         preferred_element_type=jnp.float32) + b_ref[...]


def _tc_linear(h, w, b):
    return pl.pallas_call(
        _lin_body,
        out_shape=jax.ShapeDtypeStruct((N, D), jnp.float32),
    )(h, w, b)


BE = 4000
NEB = E // BE


def _pass1_body(g_ref, a_ref, w1b_ref, sum_ref, sq_ref):
    i = pl.program_id(0)
    u = jnp.maximum(g_ref[...] + a_ref[...] * w1b_ref[...], 0.0)
    s = jnp.sum(u, axis=0, keepdims=True)
    q = jnp.sum(u * u, axis=0, keepdims=True)

    @pl.when(i == 0)
    def _():
        sum_ref[...] = s
        sq_ref[...] = q

    @pl.when(i > 0)
    def _():
        sum_ref[...] += s
        sq_ref[...] += q


def _tc_pass1(G, attr, w1b):
    return pl.pallas_call(
        _pass1_body,
        grid=(NEB,),
        in_specs=[
            pl.BlockSpec((BE, D), lambda i: (i, 0)),
            pl.BlockSpec((BE, 1), lambda i: (i, 0)),
            pl.BlockSpec((1, D), lambda i: (0, 0)),
        ],
        out_specs=[
            pl.BlockSpec((1, D), lambda i: (0, 0)),
            pl.BlockSpec((1, D), lambda i: (0, 0)),
        ],
        out_shape=[
            jax.ShapeDtypeStruct((1, D), jnp.float32),
            jax.ShapeDtypeStruct((1, D), jnp.float32),
        ],
    )(G, attr, w1b)


def _pass2_body(g_ref, a_ref, w1b_ref, al_ref, be_ref, w2_ref, b2_ref,
                v_ref, sq_ref):
    i = pl.program_id(0)
    u = jnp.maximum(g_ref[...] + a_ref[...] * w1b_ref[...], 0.0)
    w2p = w2_ref[...] * al_ref[...]
    b2p = jnp.dot(be_ref[...], w2_ref[...],
                  preferred_element_type=jnp.float32) + b2_ref[...]
    v = jnp.maximum(jnp.dot(u, w2p, preferred_element_type=jnp.float32) + b2p,
                    0.0)
    v_ref[...] = v
    q = jnp.sum(v * v, axis=0, keepdims=True)

    @pl.when(i == 0)
    def _():
        sq_ref[...] = q

    @pl.when(i > 0)
    def _():
        sq_ref[...] += q


def _tc_pass2(G, attr, w1b, alpha_col, beta_row, w2, b2):
    return pl.pallas_call(
        _pass2_body,
        grid=(NEB,),
        in_specs=[
            pl.BlockSpec((BE, D), lambda i: (i, 0)),
            pl.BlockSpec((BE, 1), lambda i: (i, 0)),
            pl.BlockSpec((1, D), lambda i: (0, 0)),
            pl.BlockSpec((D, 1), lambda i: (0, 0)),
            pl.BlockSpec((1, D), lambda i: (0, 0)),
            pl.BlockSpec((D, D), lambda i: (0, 0)),
            pl.BlockSpec((1, D), lambda i: (0, 0)),
        ],
        out_specs=[
            pl.BlockSpec((BE, D), lambda i: (i, 0)),
            pl.BlockSpec((1, D), lambda i: (0, 0)),
        ],
        out_shape=[
            jax.ShapeDtypeStruct((E, D), jnp.float32),
            jax.ShapeDtypeStruct((1, D), jnp.float32),
        ],
    )(G, attr, w1b, alpha_col, beta_row, w2, b2)


def _node_body(h_ref, s_ref, cnt_ref, q2_ref, g2_ref, be2_ref,
               wa_ref, wb_ref, b1_ref, g1_ref, bb1_ref,
               w2_ref, b2_ref, gg2_ref, bb2_ref, o_ref):
    sv = s_ref[0] + s_ref[1]
    cnt = cnt_ref[0, :, 0:1] + cnt_ref[1, :, 0:1]
    mu2 = jnp.sum(sv, axis=0, keepdims=True) / E
    var2 = q2_ref[...] / E - mu2 * mu2
    gamma = g2_ref[...] * lax.rsqrt(var2 + EPS)
    delta = be2_ref[...] - mu2 * gamma
    agg = (sv * gamma + cnt * delta) / jnp.maximum(cnt, 1.0)

    z = jnp.dot(h_ref[...], wa_ref[...], preferred_element_type=jnp.float32)
    z += jnp.dot(agg, wb_ref[...], preferred_element_type=jnp.float32)
    z = jnp.maximum(z + b1_ref[...], 0.0)
    mu = jnp.mean(z, axis=0, keepdims=True)
    var = jnp.mean(z * z, axis=0, keepdims=True) - mu * mu
    z = (z - mu) * lax.rsqrt(var + EPS) * g1_ref[...] + bb1_ref[...]

    z2 = jnp.maximum(
        jnp.dot(z, w2_ref[...], preferred_element_type=jnp.float32)
        + b2_ref[...], 0.0)
    mu = jnp.mean(z2, axis=0, keepdims=True)
    var = jnp.mean(z2 * z2, axis=0, keepdims=True) - mu * mu
    o_ref[...] = (z2 - mu) * lax.rsqrt(var + EPS) * gg2_ref[...] + bb2_ref[...]


def _tc_node(h, s_part, cnt_part, q2, g2, be2, wa, wb, b1, g1, bb1,
             w2, b2, gg2, bb2):
    return pl.pallas_call(
        _node_body,
        out_shape=jax.ShapeDtypeStruct((N, D), jnp.float32),
    )(h, s_part, cnt_part, q2, g2, be2, wa, wb, b1, g1, bb1, w2, b2, gg2, bb2)


def _row(v):
    return v.reshape(1, -1).astype(jnp.float32)


def kernel(x, edge_index, edge_attr, idx, params):
    src = edge_index[1]
    attr = edge_attr.astype(jnp.float32)

    znd = jnp.zeros((N, D), jnp.float32)
    z16 = jnp.zeros((N, 16), jnp.float32)
    ones16 = jnp.ones((CH, 16), jnp.float32)

    h = x
    for lp in params:
        m1, m2 = lp["msg"]
        n1, n2 = lp["node"]
        w1a = m1["W"][:D]
        w1b = _row(m1["W"][D])
        b1 = _row(m1["b"])

        A = _tc_linear(h, w1a, b1)
        G = _sc_gather(A, src)
        s1, q1 = _tc_pass1(G, attr, w1b)
        mu1 = s1 / E
        var1 = q1 / E - mu1 * mu1
        alpha = m1["g"].reshape(1, D) * lax.rsqrt(var1 + EPS)
        beta = _row(m1["be"]) - mu1 * alpha
        v, q2 = _tc_pass2(G, attr, w1b, alpha.reshape(D, 1), beta,
                          m2["W"], _row(m2["b"]))
        s_part, cnt_part = _sc_scatter(v, src, znd, z16, ones16)

        nout = n2["W"].shape[1]
        w2p = jnp.zeros((D, D), jnp.float32).at[:, :nout].set(n2["W"])
        b2p = jnp.zeros((1, D), jnp.float32).at[:, :nout].set(_row(n2["b"]))
        g2p = jnp.ones((1, D), jnp.float32).at[:, :nout].set(_row(n2["g"]))
        be2p = jnp.zeros((1, D), jnp.float32).at[:, :nout].set(_row(n2["be"]))

        h = _tc_node(h, s_part, cnt_part, q2, _row(m2["g"]), _row(m2["be"]),
                     n1["W"][:D], n1["W"][D:], _row(n1["b"]), _row(n1["g"]),
                     _row(n1["be"]), w2p, b2p, g2p, be2p)

    srcf = jnp.concatenate(
        [idx.reshape(-1), jnp.zeros((PP - N * K,), jnp.int32)])
    dstf = jnp.concatenate(
        [jnp.repeat(jnp.arange(N, dtype=jnp.int32), K),
         jnp.zeros((PP - N * K,), jnp.int32)])
    dists = _sc_dist(h[:, 0], h[:, 1], srcf, dstf)
    return dists[:N * K].reshape(-1, 1)


# speedup vs baseline: 3.1859x; 3.1859x over previous
import functools

import jax
import jax.numpy as jnp
from jax import lax
from jax.experimental import pallas as pl
from jax.experimental.pallas import tpu as pltpu
from jax.experimental.pallas import tpu_sc as plsc

N = 10000
K = 17
D = 128
E = N * (K - 1)
EPS = 1e-5

NC, NS = 2, 16
NW = NC * NS
CH = 128
NCHUNK = E // CH
CPW = -(-NCHUNK // NW)
RPT = 632

PT = 5328
PP = PT * NW
DLOOP = PT // 16

def _mesh():
    return plsc.VectorSubcoreMesh(
        core_axis_name="c", subcore_axis_name="s",
        num_cores=NC, num_subcores=NS)


@functools.cache
def _sc_gather_kernel():
    @functools.partial(
        pl.kernel,
        out_type=jax.ShapeDtypeStruct((E, D), jnp.float32),
        mesh=_mesh(),
        scratch_types=[
            pltpu.VMEM((CH,), jnp.int32),
            pltpu.VMEM((CH, D), jnp.float32),
            pltpu.SemaphoreType.DMA,
        ],
    )
    def body(table_hbm, src_hbm, out_hbm, idx_v, rows_v, sem):
        wid = lax.axis_index("s") * NC + lax.axis_index("c")
        for i in range(CPW):
            cid = wid + i * NW

            @pl.when(cid < NCHUNK)
            def _():
                base = cid * CH
                pltpu.sync_copy(src_hbm.at[pl.ds(base, CH)], idx_v)
                pltpu.async_copy(table_hbm.at[idx_v], rows_v, sem).wait()
                pltpu.sync_copy(rows_v, out_hbm.at[pl.ds(base, CH)])

    return body


def _sc_gather(table, src):
    return _sc_gather_kernel()(table, src)


@functools.cache
def _sc_scatter_kernel(with_cnt):
    outs = [jax.ShapeDtypeStruct((NC, N, D), jnp.float32)]
    if with_cnt:
        outs.append(jax.ShapeDtypeStruct((NC, N, D), jnp.float32))

    @functools.partial(
        pl.kernel,
        out_type=tuple(outs) if with_cnt else outs[0],
        mesh=_mesh(),
        scratch_types=[
            pltpu.VMEM((CH,), jnp.int32),
            pltpu.VMEM((CH, D), jnp.float32),
            pltpu.VMEM_SHARED((N, D), jnp.float32),
        ],
    )
    def body(v_hbm, src_hbm, *refs):
        if with_cnt:
            s_out, cnt_out, idx_v, rows_v, s_sh = refs
        else:
            s_out, idx_v, rows_v, s_sh = refs
        c = lax.axis_index("c")
        s = lax.axis_index("s")
        wid = s * NC + c
        row0 = jnp.minimum(s * RPT, N - RPT)
        pieces = []
        off = 0
        while off < RPT:
            pieces.append((off, min(CH, RPT - off)))
            off += CH
        zeros16 = jnp.zeros((16,), jnp.float32)
        ones16 = jnp.ones((16,), jnp.float32)

        def fill(val):
            def frow(r, carry):
                for j in range(D // 16):
                    rows_v[r, pl.ds(j * 16, 16)] = val
                return carry
            lax.fori_loop(0, CH, frow, 0)

        def accumulate(load_rows):
            fill(zeros16)
            for off, sz in pieces:
                pltpu.sync_copy(rows_v.at[pl.ds(0, sz)],
                                s_sh.at[pl.ds(row0 + off, sz)])
            if not load_rows:
                fill(ones16)
            plsc.subcore_barrier()
            for i in range(CPW):
                cid = wid + i * NW

                @pl.when(cid < NCHUNK)
                def _():
                    base = cid * CH
                    pltpu.sync_copy(src_hbm.at[pl.ds(base, CH)], idx_v)
                    if load_rows:
                        pltpu.sync_copy(v_hbm.at[pl.ds(base, CH)], rows_v)
                    pltpu.sync_copy(rows_v, s_sh.at[idx_v], add=True)
            plsc.subcore_barrier()

        def drain(out_ref):
            for off, sz in pieces:
                pltpu.sync_copy(s_sh.at[pl.ds(row0 + off, sz)],
                                rows_v.at[pl.ds(0, sz)])
                pltpu.sync_copy(rows_v.at[pl.ds(0, sz)],
                                out_ref.at[c, pl.ds(row0 + off, sz)])

        accumulate(True)
        drain(s_out)
        if with_cnt:
            plsc.subcore_barrier()
            accumulate(False)
            drain(cnt_out)

    return body


def _sc_scatter(v, src, with_cnt):
    return _sc_scatter_kernel(with_cnt)(v, src)


@functools.cache
def _sc_dist_kernel():
    @functools.partial(
        pl.kernel,
        out_type=jax.ShapeDtypeStruct((PP,), jnp.float32),
        mesh=_mesh(),
        compiler_params=pltpu.CompilerParams(needs_layout_passes=False),
        scratch_types=[
            pltpu.VMEM((N,), jnp.float32),
            pltpu.VMEM((N,), jnp.float32),
            pltpu.VMEM((PT,), jnp.int32),
            pltpu.VMEM((PT,), jnp.int32),
            pltpu.VMEM((PT,), jnp.float32),
        ],
    )
    def body(h0_hbm, h1_hbm, srcf_hbm, dstf_hbm, out_hbm,
             c0, c1, si, di, ob):
        wid = lax.axis_index("s") * NC + lax.axis_index("c")
        base = wid * PT
        pltpu.sync_copy(h0_hbm, c0)
        pltpu.sync_copy(h1_hbm, c1)
        pltpu.sync_copy(srcf_hbm.at[pl.ds(base, PT)], si)
        pltpu.sync_copy(dstf_hbm.at[pl.ds(base, PT)], di)

        def step(j, carry):
            sv = si[pl.ds(j * 16, 16)]
            dv = di[pl.ds(j * 16, 16)]
            a0 = plsc.load_gather(c0, [sv])
            a1 = plsc.load_gather(c1, [sv])
            b0 = plsc.load_gather(c0, [dv])
            b1 = plsc.load_gather(c1, [dv])
            d0 = b0 - a0
            d1 = b1 - a1
            ob[pl.ds(j * 16, 16)] = d0 * d0 + d1 * d1
            return carry

        lax.fori_loop(0, DLOOP, step, 0)
        pltpu.sync_copy(ob, out_hbm.at[pl.ds(base, PT)])

    return body


def _sc_dist(h0, h1, srcf, dstf):
    return _sc_dist_kernel()(h0, h1, srcf, dstf)


def _lin_body(h_ref, w_ref, o_ref):
    o_ref[...] = jnp.dot(h_ref[...], w_ref[...],
                         preferred_element_type=jnp.float32)


def _tc_linear(h, w):
    return pl.pallas_call(
        _lin_body,
        out_shape=jax.ShapeDtypeStruct((N, D), jnp.float32),
    )(h, w)


BE = 4000
NEB = E // BE


def _pass1_body(g_ref, a_ref, w1b_ref, b1_ref, sum_ref, sq_ref):
    i = pl.program_id(0)
    ab = a_ref[...].astype(jnp.bfloat16).astype(jnp.float32)
    wb = w1b_ref[...].astype(jnp.bfloat16).astype(jnp.float32)
    u = jnp.maximum((g_ref[...] + ab * wb) + b1_ref[...], 0.0)
    s = jnp.sum(u, axis=0, keepdims=True)
    q = jnp.sum(u * u, axis=0, keepdims=True)

    @pl.when(i == 0)
    def _():
        sum_ref[...] = s
        sq_ref[...] = q

    @pl.when(i > 0)
    def _():
        sum_ref[...] += s
        sq_ref[...] += q


def _tc_pass1(G, attr, w1b, b1):
    return pl.pallas_call(
        _pass1_body,
        grid=(NEB,),
        in_specs=[
            pl.BlockSpec((BE, D), lambda i: (i, 0)),
            pl.BlockSpec((BE, 1), lambda i: (i, 0)),
            pl.BlockSpec((1, D), lambda i: (0, 0)),
            pl.BlockSpec((1, D), lambda i: (0, 0)),
        ],
        out_specs=[
            pl.BlockSpec((1, D), lambda i: (0, 0)),
            pl.BlockSpec((1, D), lambda i: (0, 0)),
        ],
        out_shape=[
            jax.ShapeDtypeStruct((1, D), jnp.float32),
            jax.ShapeDtypeStruct((1, D), jnp.float32),
        ],
    )(G, attr, w1b, b1)


def _pass2_body(g_ref, a_ref, w1b_ref, b1_ref, mu_ref, var_ref, g1_ref,
                be1_ref, w2_ref, b2_ref, v_ref, sq_ref):
    i = pl.program_id(0)
    ab = a_ref[...].astype(jnp.bfloat16).astype(jnp.float32)
    wb = w1b_ref[...].astype(jnp.bfloat16).astype(jnp.float32)
    u = jnp.maximum((g_ref[...] + ab * wb) + b1_ref[...], 0.0)
    uh = ((u - mu_ref[...]) / jnp.sqrt(var_ref[...] + EPS)
          * g1_ref[...] + be1_ref[...])
    v = jnp.maximum(
        jnp.dot(uh, w2_ref[...], preferred_element_type=jnp.float32)
        + b2_ref[...], 0.0)
    v_ref[...] = v
    q = jnp.sum(v * v, axis=0, keepdims=True)

    @pl.when(i == 0)
    def _():
        sq_ref[...] = q

    @pl.when(i > 0)
    def _():
        sq_ref[...] += q


def _tc_pass2(G, attr, w1b, b1, mu1, var1, g1, be1, w2, b2):
    return pl.pallas_call(
        _pass2_body,
        grid=(NEB,),
        in_specs=[
            pl.BlockSpec((BE, D), lambda i: (i, 0)),
            pl.BlockSpec((BE, 1), lambda i: (i, 0)),
            pl.BlockSpec((1, D), lambda i: (0, 0)),
            pl.BlockSpec((1, D), lambda i: (0, 0)),
            pl.BlockSpec((1, D), lambda i: (0, 0)),
            pl.BlockSpec((1, D), lambda i: (0, 0)),
            pl.BlockSpec((1, D), lambda i: (0, 0)),
            pl.BlockSpec((1, D), lambda i: (0, 0)),
            pl.BlockSpec((D, D), lambda i: (0, 0)),
            pl.BlockSpec((1, D), lambda i: (0, 0)),
        ],
        out_specs=[
            pl.BlockSpec((BE, D), lambda i: (i, 0)),
            pl.BlockSpec((1, D), lambda i: (0, 0)),
        ],
        out_shape=[
            jax.ShapeDtypeStruct((E, D), jnp.float32),
            jax.ShapeDtypeStruct((1, D), jnp.float32),
        ],
    )(G, attr, w1b, b1, mu1, var1, g1, be1, w2, b2)


def _node_body(h_ref, s_ref, cnt_ref, q2_ref, g2_ref, be2_ref,
               wa_ref, wb_ref, b1_ref, g1_ref, bb1_ref,
               w2_ref, b2_ref, gg2_ref, bb2_ref, o_ref):
    sv = s_ref[0] + s_ref[1]
    cnt = cnt_ref[0, :, 0:1] + cnt_ref[1, :, 0:1]
    mu2 = jnp.sum(sv, axis=0, keepdims=True) / E
    var2 = q2_ref[...] / E - mu2 * mu2
    gamma = g2_ref[...] / jnp.sqrt(var2 + EPS)
    delta = be2_ref[...] - mu2 * gamma
    agg = (sv * gamma + cnt * delta) / jnp.maximum(cnt, 1.0)

    z = jnp.dot(h_ref[...], wa_ref[...],
                 preferred_element_type=jnp.float32)
    z += jnp.dot(agg, wb_ref[...],
                 preferred_element_type=jnp.float32)
    z = jnp.maximum(z + b1_ref[...], 0.0)
    mu = jnp.mean(z, axis=0, keepdims=True)
    var = jnp.mean((z - mu) ** 2, axis=0, keepdims=True)
    z = (z - mu) / jnp.sqrt(var + EPS) * g1_ref[...] + bb1_ref[...]

    z2 = jnp.maximum(
        jnp.dot(z, w2_ref[...],
                 preferred_element_type=jnp.float32)
        + b2_ref[...], 0.0)
    mu = jnp.mean(z2, axis=0, keepdims=True)
    var = jnp.mean((z2 - mu) ** 2, axis=0, keepdims=True)
    o_ref[...] = (z2 - mu) / jnp.sqrt(var + EPS) * gg2_ref[...] + bb2_ref[...]


def _tc_node(h, s_part, cnt_part, q2, g2, be2, wa, wb, b1, g1, bb1,
             w2, b2, gg2, bb2):
    return pl.pallas_call(
        _node_body,
        out_shape=jax.ShapeDtypeStruct((N, D), jnp.float32),
    )(h, s_part, cnt_part, q2, g2, be2, wa, wb, b1, g1, bb1, w2, b2, gg2, bb2)


def _row(v):
    return v.reshape(1, -1).astype(jnp.float32)


def kernel(x, edge_index, edge_attr, idx, params):
    src = edge_index[1]
    attr = edge_attr.astype(jnp.float32)

    h = x
    cnt_part = None
    for lp in params:
        m1, m2 = lp["msg"]
        n1, n2 = lp["node"]
        w1a = m1["W"][:D]
        w1b = _row(m1["W"][D])
        b1 = _row(m1["b"])

        A = _tc_linear(h, w1a)
        G = _sc_gather(A, src)
        s1, q1 = _tc_pass1(G, attr, w1b, b1)
        mu1 = s1 / E
        var1 = q1 / E - mu1 * mu1
        v, q2 = _tc_pass2(G, attr, w1b, b1, mu1, var1, _row(m1["g"]),
                          _row(m1["be"]), m2["W"], _row(m2["b"]))
        if cnt_part is None:
            s_part, cnt_part = _sc_scatter(v, src, True)
        else:
            s_part = _sc_scatter(v, src, False)

        nout = n2["W"].shape[1]
        w2p = jnp.zeros((D, D), jnp.float32).at[:, :nout].set(n2["W"])
        b2p = jnp.zeros((1, D), jnp.float32).at[:, :nout].set(_row(n2["b"]))
        g2p = jnp.ones((1, D), jnp.float32).at[:, :nout].set(_row(n2["g"]))
        be2p = jnp.zeros((1, D), jnp.float32).at[:, :nout].set(_row(n2["be"]))

        h = _tc_node(h, s_part, cnt_part, q2, _row(m2["g"]), _row(m2["be"]),
                     n1["W"][:D], n1["W"][D:], _row(n1["b"]), _row(n1["g"]),
                     _row(n1["be"]), w2p, b2p, g2p, be2p)

    srcf = jnp.concatenate(
        [idx.reshape(-1), jnp.zeros((PP - N * K,), jnp.int32)])
    dstf = jnp.concatenate(
        [jnp.repeat(jnp.arange(N, dtype=jnp.int32), K),
         jnp.zeros((PP - N * K,), jnp.int32)])
    dists = _sc_dist(h[:, 0], h[:, 1], srcf, dstf)
    return dists[:N * K].reshape(-1, 1)


# speedup vs baseline: 3.8482x; 1.2079x over previous
import functools

import jax
import jax.numpy as jnp
from jax import lax
from jax.experimental import pallas as pl
from jax.experimental.pallas import tpu as pltpu
from jax.experimental.pallas import tpu_sc as plsc

N = 10000
K = 17
D = 128
E = N * (K - 1)
EPS = 1e-5

NC, NS = 2, 16
NW = NC * NS
CH = 128
NCHUNK = E // CH
CPW = -(-NCHUNK // NW)
RPT = 632

PT = 5328
PP = PT * NW
DLOOP = PT // 16

def _mesh():
    return plsc.VectorSubcoreMesh(
        core_axis_name="c", subcore_axis_name="s",
        num_cores=NC, num_subcores=NS)


@functools.cache
def _sc_gather_kernel():
    @functools.partial(
        pl.kernel,
        out_type=jax.ShapeDtypeStruct((E, D), jnp.float32),
        mesh=_mesh(),
        scratch_types=[
            pltpu.VMEM((CH,), jnp.int32),
            pltpu.VMEM((CH,), jnp.int32),
            pltpu.VMEM((CH, D), jnp.float32),
            pltpu.VMEM((CH, D), jnp.float32),
            pltpu.SemaphoreType.DMA,
            pltpu.SemaphoreType.DMA,
        ],
    )
    def body(table_hbm, src_hbm, out_hbm, idx0, idx1, rows0, rows1,
             sem0, sem1):
        idxs = (idx0, idx1)
        rows = (rows0, rows1)
        sems = (sem0, sem1)
        wid = lax.axis_index("s") * NC + lax.axis_index("c")
        for i in range(CPW + 1):
            b = i & 1
            if i < CPW:
                cid = wid + i * NW

                @pl.when(cid < NCHUNK)
                def _():
                    base = cid * CH
                    pltpu.sync_copy(src_hbm.at[pl.ds(base, CH)], idxs[b])
                    pltpu.async_copy(table_hbm.at[idxs[b]], rows[b], sems[b])
            if i > 0:
                pcid = wid + (i - 1) * NW

                @pl.when(pcid < NCHUNK)
                def _():
                    pltpu.make_async_copy(
                        table_hbm.at[pl.ds(0, CH)], rows[1 - b],
                        sems[1 - b]).wait()
                    pltpu.sync_copy(rows[1 - b],
                                    out_hbm.at[pl.ds(pcid * CH, CH)])

    return body


def _sc_gather(table, src):
    return _sc_gather_kernel()(table, src)


@functools.cache
def _sc_scatter_kernel(with_cnt):
    outs = [jax.ShapeDtypeStruct((NC, N, D), jnp.float32)]
    if with_cnt:
        outs.append(jax.ShapeDtypeStruct((NC, N, D), jnp.float32))

    @functools.partial(
        pl.kernel,
        out_type=tuple(outs) if with_cnt else outs[0],
        mesh=_mesh(),
        scratch_types=[
            pltpu.VMEM((CH,), jnp.int32),
            pltpu.VMEM((CH,), jnp.int32),
            pltpu.VMEM((CH, D), jnp.float32),
            pltpu.VMEM((CH, D), jnp.float32),
            pltpu.SemaphoreType.DMA,
            pltpu.SemaphoreType.DMA,
            pltpu.VMEM_SHARED((N, D), jnp.float32),
        ],
    )
    def body(v_hbm, src_hbm, *refs):
        if with_cnt:
            s_out, cnt_out, idx0, idx1, rows0, rows1, sem0, sem1, s_sh = refs
        else:
            s_out, idx0, idx1, rows0, rows1, sem0, sem1, s_sh = refs
        idxs = (idx0, idx1)
        rows = (rows0, rows1)
        sems = (sem0, sem1)
        idx_v, rows_v = idx0, rows0
        c = lax.axis_index("c")
        s = lax.axis_index("s")
        wid = s * NC + c
        row0 = jnp.minimum(s * RPT, N - RPT)
        pieces = []
        off = 0
        while off < RPT:
            pieces.append((off, min(CH, RPT - off)))
            off += CH
        zeros16 = jnp.zeros((16,), jnp.float32)
        ones16 = jnp.ones((16,), jnp.float32)

        def fill(val):
            def frow(r, carry):
                for j in range(D // 16):
                    rows_v[r, pl.ds(j * 16, 16)] = val
                return carry
            lax.fori_loop(0, CH, frow, 0)

        def zero_accum():
            fill(zeros16)
            for off, sz in pieces:
                pltpu.sync_copy(rows_v.at[pl.ds(0, sz)],
                                s_sh.at[pl.ds(row0 + off, sz)])

        def accumulate(load_rows):
            zero_accum()
            if not load_rows:
                fill(ones16)
            plsc.subcore_barrier()
            for i in range(CPW + 1):
                b = i & 1
                if i < CPW:
                    cid = wid + i * NW

                    @pl.when(cid < NCHUNK)
                    def _():
                        base = cid * CH
                        pltpu.sync_copy(src_hbm.at[pl.ds(base, CH)], idxs[b])
                        if load_rows:
                            pltpu.async_copy(v_hbm.at[pl.ds(base, CH)],
                                             rows[b], sems[b])
                if i > 0:
                    pb = 1 - b
                    pcid = wid + (i - 1) * NW

                    @pl.when(pcid < NCHUNK)
                    def _():
                        if load_rows:
                            pltpu.make_async_copy(
                                v_hbm.at[pl.ds(0, CH)], rows[pb],
                                sems[pb]).wait()
                            pltpu.sync_copy(rows[pb], s_sh.at[idxs[pb]],
                                            add=True)
                        else:
                            pltpu.sync_copy(rows_v, s_sh.at[idxs[pb]],
                                            add=True)
            plsc.subcore_barrier()

        def drain(out_ref):
            for off, sz in pieces:
                pltpu.sync_copy(s_sh.at[pl.ds(row0 + off, sz)],
                                rows_v.at[pl.ds(0, sz)])
                pltpu.sync_copy(rows_v.at[pl.ds(0, sz)],
                                out_ref.at[c, pl.ds(row0 + off, sz)])

        accumulate(True)
        drain(s_out)
        if with_cnt:
            plsc.subcore_barrier()
            accumulate(False)
            drain(cnt_out)

    return body


def _sc_scatter(v, src, with_cnt):
    return _sc_scatter_kernel(with_cnt)(v, src)


@functools.cache
def _sc_dist_kernel():
    @functools.partial(
        pl.kernel,
        out_type=jax.ShapeDtypeStruct((PP,), jnp.float32),
        mesh=_mesh(),
        compiler_params=pltpu.CompilerParams(needs_layout_passes=False),
        scratch_types=[
            pltpu.VMEM((N,), jnp.float32),
            pltpu.VMEM((N,), jnp.float32),
            pltpu.VMEM((PT,), jnp.int32),
            pltpu.VMEM((PT,), jnp.int32),
            pltpu.VMEM((PT,), jnp.float32),
        ],
    )
    def body(h0_hbm, h1_hbm, srcf_hbm, dstf_hbm, out_hbm,
             c0, c1, si, di, ob):
        wid = lax.axis_index("s") * NC + lax.axis_index("c")
        base = wid * PT
        pltpu.sync_copy(h0_hbm, c0)
        pltpu.sync_copy(h1_hbm, c1)
        pltpu.sync_copy(srcf_hbm.at[pl.ds(base, PT)], si)
        pltpu.sync_copy(dstf_hbm.at[pl.ds(base, PT)], di)

        def step(j, carry):
            sv = si[pl.ds(j * 16, 16)]
            dv = di[pl.ds(j * 16, 16)]
            a0 = plsc.load_gather(c0, [sv])
            a1 = plsc.load_gather(c1, [sv])
            b0 = plsc.load_gather(c0, [dv])
            b1 = plsc.load_gather(c1, [dv])
            d0 = b0 - a0
            d1 = b1 - a1
            ob[pl.ds(j * 16, 16)] = d0 * d0 + d1 * d1
            return carry

        lax.fori_loop(0, DLOOP, step, 0)
        pltpu.sync_copy(ob, out_hbm.at[pl.ds(base, PT)])

    return body


def _sc_dist(h0, h1, srcf, dstf):
    return _sc_dist_kernel()(h0, h1, srcf, dstf)


def _lin_body(h_ref, w_ref, o_ref):
    o_ref[...] = jnp.dot(h_ref[...], w_ref[...],
                         preferred_element_type=jnp.float32)


def _tc_linear(h, w):
    return pl.pallas_call(
        _lin_body,
        out_shape=jax.ShapeDtypeStruct((N, D), jnp.float32),
    )(h, w)


BE = 4000
NEB = E // BE


def _pass1_body(g_ref, a_ref, w1b_ref, b1_ref, sum_ref, sq_ref):
    i = pl.program_id(0)
    ab = a_ref[...].astype(jnp.bfloat16).astype(jnp.float32)
    wb = w1b_ref[...].astype(jnp.bfloat16).astype(jnp.float32)
    u = jnp.maximum((g_ref[...] + ab * wb) + b1_ref[...], 0.0)
    s = jnp.sum(u, axis=0, keepdims=True)
    q = jnp.sum(u * u, axis=0, keepdims=True)

    @pl.when(i == 0)
    def _():
        sum_ref[...] = s
        sq_ref[...] = q

    @pl.when(i > 0)
    def _():
        sum_ref[...] += s
        sq_ref[...] += q


def _tc_pass1(G, attr, w1b, b1):
    return pl.pallas_call(
        _pass1_body,
        grid=(NEB,),
        in_specs=[
            pl.BlockSpec((BE, D), lambda i: (i, 0)),
            pl.BlockSpec((BE, 1), lambda i: (i, 0)),
            pl.BlockSpec((1, D), lambda i: (0, 0)),
            pl.BlockSpec((1, D), lambda i: (0, 0)),
        ],
        out_specs=[
            pl.BlockSpec((1, D), lambda i: (0, 0)),
            pl.BlockSpec((1, D), lambda i: (0, 0)),
        ],
        out_shape=[
            jax.ShapeDtypeStruct((1, D), jnp.float32),
            jax.ShapeDtypeStruct((1, D), jnp.float32),
        ],
    )(G, attr, w1b, b1)


def _pass2_body(g_ref, a_ref, w1b_ref, b1_ref, mu_ref, var_ref, g1_ref,
                be1_ref, w2_ref, b2_ref, v_ref, sq_ref):
    i = pl.program_id(0)
    ab = a_ref[...].astype(jnp.bfloat16).astype(jnp.float32)
    wb = w1b_ref[...].astype(jnp.bfloat16).astype(jnp.float32)
    u = jnp.maximum((g_ref[...] + ab * wb) + b1_ref[...], 0.0)
    uh = ((u - mu_ref[...]) / jnp.sqrt(var_ref[...] + EPS)
          * g1_ref[...] + be1_ref[...])
    v = jnp.maximum(
        jnp.dot(uh, w2_ref[...], preferred_element_type=jnp.float32)
        + b2_ref[...], 0.0)
    v_ref[...] = v
    q = jnp.sum(v * v, axis=0, keepdims=True)

    @pl.when(i == 0)
    def _():
        sq_ref[...] = q

    @pl.when(i > 0)
    def _():
        sq_ref[...] += q


def _tc_pass2(G, attr, w1b, b1, mu1, var1, g1, be1, w2, b2):
    return pl.pallas_call(
        _pass2_body,
        grid=(NEB,),
        in_specs=[
            pl.BlockSpec((BE, D), lambda i: (i, 0)),
            pl.BlockSpec((BE, 1), lambda i: (i, 0)),
            pl.BlockSpec((1, D), lambda i: (0, 0)),
            pl.BlockSpec((1, D), lambda i: (0, 0)),
            pl.BlockSpec((1, D), lambda i: (0, 0)),
            pl.BlockSpec((1, D), lambda i: (0, 0)),
            pl.BlockSpec((1, D), lambda i: (0, 0)),
            pl.BlockSpec((1, D), lambda i: (0, 0)),
            pl.BlockSpec((D, D), lambda i: (0, 0)),
            pl.BlockSpec((1, D), lambda i: (0, 0)),
        ],
        out_specs=[
            pl.BlockSpec((BE, D), lambda i: (i, 0)),
            pl.BlockSpec((1, D), lambda i: (0, 0)),
        ],
        out_shape=[
            jax.ShapeDtypeStruct((E, D), jnp.float32),
            jax.ShapeDtypeStruct((1, D), jnp.float32),
        ],
    )(G, attr, w1b, b1, mu1, var1, g1, be1, w2, b2)


def _node_body(h_ref, s_ref, cnt_ref, q2_ref, g2_ref, be2_ref,
               wa_ref, wb_ref, b1_ref, g1_ref, bb1_ref,
               w2_ref, b2_ref, gg2_ref, bb2_ref, o_ref):
    sv = s_ref[0] + s_ref[1]
    cnt = cnt_ref[0, :, 0:1] + cnt_ref[1, :, 0:1]
    mu2 = jnp.sum(sv, axis=0, keepdims=True) / E
    var2 = q2_ref[...] / E - mu2 * mu2
    gamma = g2_ref[...] / jnp.sqrt(var2 + EPS)
    delta = be2_ref[...] - mu2 * gamma
    agg = (sv * gamma + cnt * delta) / jnp.maximum(cnt, 1.0)

    z = jnp.dot(h_ref[...], wa_ref[...],
                 preferred_element_type=jnp.float32)
    z += jnp.dot(agg, wb_ref[...],
                 preferred_element_type=jnp.float32)
    z = jnp.maximum(z + b1_ref[...], 0.0)
    mu = jnp.mean(z, axis=0, keepdims=True)
    var = jnp.mean((z - mu) ** 2, axis=0, keepdims=True)
    z = (z - mu) / jnp.sqrt(var + EPS) * g1_ref[...] + bb1_ref[...]

    z2 = jnp.maximum(
        jnp.dot(z, w2_ref[...],
                 preferred_element_type=jnp.float32)
        + b2_ref[...], 0.0)
    mu = jnp.mean(z2, axis=0, keepdims=True)
    var = jnp.mean((z2 - mu) ** 2, axis=0, keepdims=True)
    o_ref[...] = (z2 - mu) / jnp.sqrt(var + EPS) * gg2_ref[...] + bb2_ref[...]


def _tc_node(h, s_part, cnt_part, q2, g2, be2, wa, wb, b1, g1, bb1,
             w2, b2, gg2, bb2):
    return pl.pallas_call(
        _node_body,
        out_shape=jax.ShapeDtypeStruct((N, D), jnp.float32),
    )(h, s_part, cnt_part, q2, g2, be2, wa, wb, b1, g1, bb1, w2, b2, gg2, bb2)


def _row(v):
    return v.reshape(1, -1).astype(jnp.float32)


def kernel(x, edge_index, edge_attr, idx, params):
    src = edge_index[1]
    attr = edge_attr.astype(jnp.float32)

    h = x
    cnt_part = None
    for lp in params:
        m1, m2 = lp["msg"]
        n1, n2 = lp["node"]
        w1a = m1["W"][:D]
        w1b = _row(m1["W"][D])
        b1 = _row(m1["b"])

        A = _tc_linear(h, w1a)
        G = _sc_gather(A, src)
        s1, q1 = _tc_pass1(G, attr, w1b, b1)
        mu1 = s1 / E
        var1 = q1 / E - mu1 * mu1
        v, q2 = _tc_pass2(G, attr, w1b, b1, mu1, var1, _row(m1["g"]),
                          _row(m1["be"]), m2["W"], _row(m2["b"]))
        if cnt_part is None:
            s_part, cnt_part = _sc_scatter(v, src, True)
        else:
            s_part = _sc_scatter(v, src, False)

        nout = n2["W"].shape[1]
        w2p = jnp.zeros((D, D), jnp.float32).at[:, :nout].set(n2["W"])
        b2p = jnp.zeros((1, D), jnp.float32).at[:, :nout].set(_row(n2["b"]))
        g2p = jnp.ones((1, D), jnp.float32).at[:, :nout].set(_row(n2["g"]))
        be2p = jnp.zeros((1, D), jnp.float32).at[:, :nout].set(_row(n2["be"]))

        h = _tc_node(h, s_part, cnt_part, q2, _row(m2["g"]), _row(m2["be"]),
                     n1["W"][:D], n1["W"][D:], _row(n1["b"]), _row(n1["g"]),
                     _row(n1["be"]), w2p, b2p, g2p, be2p)

    srcf = jnp.concatenate(
        [idx.reshape(-1), jnp.zeros((PP - N * K,), jnp.int32)])
    dstf = jnp.concatenate(
        [jnp.repeat(jnp.arange(N, dtype=jnp.int32), K),
         jnp.zeros((PP - N * K,), jnp.int32)])
    dists = _sc_dist(h[:, 0], h[:, 1], srcf, dstf)
    return dists[:N * K].reshape(-1, 1)


# speedup vs baseline: 3.8837x; 1.0092x over previous
import functools

import jax
import jax.numpy as jnp
from jax import lax
from jax.experimental import pallas as pl
from jax.experimental.pallas import tpu as pltpu
from jax.experimental.pallas import tpu_sc as plsc

N = 10000
K = 17
D = 128
E = N * (K - 1)
EPS = 1e-5

NC, NS = 2, 16
NW = NC * NS
CH = 128
NCHUNK = E // CH
CPW = -(-NCHUNK // NW)
RPT = 632

PT = 5328
PP = PT * NW
DLOOP = PT // 16

def _mesh():
    return plsc.VectorSubcoreMesh(
        core_axis_name="c", subcore_axis_name="s",
        num_cores=NC, num_subcores=NS)


@functools.cache
def _sc_gather_kernel():
    @functools.partial(
        pl.kernel,
        out_type=jax.ShapeDtypeStruct((E, D), jnp.float32),
        mesh=_mesh(),
        scratch_types=[
            pltpu.VMEM((CH,), jnp.int32),
            pltpu.VMEM((CH,), jnp.int32),
            pltpu.VMEM((CH,), jnp.int32),
            pltpu.VMEM((CH, D), jnp.float32),
            pltpu.VMEM((CH, D), jnp.float32),
            pltpu.VMEM((CH, D), jnp.float32),
            pltpu.SemaphoreType.DMA,
            pltpu.SemaphoreType.DMA,
            pltpu.SemaphoreType.DMA,
        ],
    )
    def body(table_hbm, src_hbm, out_hbm, idx0, idx1, idx2,
             rows0, rows1, rows2, sem0, sem1, sem2):
        idxs = (idx0, idx1, idx2)
        rows = (rows0, rows1, rows2)
        sems = (sem0, sem1, sem2)
        wid = lax.axis_index("s") * NC + lax.axis_index("c")
        for i in range(CPW + 2):
            b = i % 3
            if i < CPW:
                cid = wid + i * NW

                @pl.when(cid < NCHUNK)
                def _():
                    base = cid * CH
                    pltpu.sync_copy(src_hbm.at[pl.ds(base, CH)], idxs[b])
                    pltpu.async_copy(table_hbm.at[idxs[b]], rows[b], sems[b])
            if i > 1:
                pb = (i - 2) % 3
                pcid = wid + (i - 2) * NW

                @pl.when(pcid < NCHUNK)
                def _():
                    pltpu.make_async_copy(
                        table_hbm.at[pl.ds(0, CH)], rows[pb],
                        sems[pb]).wait()
                    pltpu.sync_copy(rows[pb],
                                    out_hbm.at[pl.ds(pcid * CH, CH)])

    return body


def _sc_gather(table, src):
    return _sc_gather_kernel()(table, src)


@functools.cache
def _sc_scatter_kernel(with_cnt):
    outs = [jax.ShapeDtypeStruct((NC, N, D), jnp.float32)]
    if with_cnt:
        outs.append(jax.ShapeDtypeStruct((NC, N, D), jnp.float32))

    @functools.partial(
        pl.kernel,
        out_type=tuple(outs) if with_cnt else outs[0],
        mesh=_mesh(),
        scratch_types=[
            pltpu.VMEM((CH,), jnp.int32),
            pltpu.VMEM((CH,), jnp.int32),
            pltpu.VMEM((CH, D), jnp.float32),
            pltpu.VMEM((CH, D), jnp.float32),
            pltpu.SemaphoreType.DMA,
            pltpu.SemaphoreType.DMA,
            pltpu.VMEM_SHARED((N, D), jnp.float32),
        ],
    )
    def body(v_hbm, src_hbm, *refs):
        if with_cnt:
            s_out, cnt_out, idx0, idx1, rows0, rows1, sem0, sem1, s_sh = refs
        else:
            s_out, idx0, idx1, rows0, rows1, sem0, sem1, s_sh = refs
        idxs = (idx0, idx1)
        rows = (rows0, rows1)
        sems = (sem0, sem1)
        idx_v, rows_v = idx0, rows0
        c = lax.axis_index("c")
        s = lax.axis_index("s")
        wid = s * NC + c
        row0 = jnp.minimum(s * RPT, N - RPT)
        pieces = []
        off = 0
        while off < RPT:
            pieces.append((off, min(CH, RPT - off)))
            off += CH
        zeros16 = jnp.zeros((16,), jnp.float32)
        ones16 = jnp.ones((16,), jnp.float32)

        def fill(val):
            def frow(r, carry):
                for j in range(D // 16):
                    rows_v[r, pl.ds(j * 16, 16)] = val
                return carry
            lax.fori_loop(0, CH, frow, 0)

        def zero_accum():
            fill(zeros16)
            for off, sz in pieces:
                pltpu.sync_copy(rows_v.at[pl.ds(0, sz)],
                                s_sh.at[pl.ds(row0 + off, sz)])

        def accumulate(load_rows):
            zero_accum()
            if not load_rows:
                fill(ones16)
            plsc.subcore_barrier()
            for i in range(CPW + 1):
                b = i & 1
                if i < CPW:
                    cid = wid + i * NW

                    @pl.when(cid < NCHUNK)
                    def _():
                        base = cid * CH
                        pltpu.sync_copy(src_hbm.at[pl.ds(base, CH)], idxs[b])
                        if load_rows:
                            pltpu.async_copy(v_hbm.at[pl.ds(base, CH)],
                                             rows[b], sems[b])
                if i > 0:
                    pb = 1 - b
                    pcid = wid + (i - 1) * NW

                    @pl.when(pcid < NCHUNK)
                    def _():
                        if load_rows:
                            pltpu.make_async_copy(
                                v_hbm.at[pl.ds(0, CH)], rows[pb],
                                sems[pb]).wait()
                            pltpu.sync_copy(rows[pb], s_sh.at[idxs[pb]],
                                            add=True)
                        else:
                            pltpu.sync_copy(rows_v, s_sh.at[idxs[pb]],
                                            add=True)
            plsc.subcore_barrier()

        def drain(out_ref):
            for off, sz in pieces:
                pltpu.sync_copy(s_sh.at[pl.ds(row0 + off, sz)],
                                rows_v.at[pl.ds(0, sz)])
                pltpu.sync_copy(rows_v.at[pl.ds(0, sz)],
                                out_ref.at[c, pl.ds(row0 + off, sz)])

        accumulate(True)
        drain(s_out)
        if with_cnt:
            plsc.subcore_barrier()
            accumulate(False)
            drain(cnt_out)

    return body


def _sc_scatter(v, src, with_cnt):
    return _sc_scatter_kernel(with_cnt)(v, src)


@functools.cache
def _sc_dist_kernel():
    @functools.partial(
        pl.kernel,
        out_type=jax.ShapeDtypeStruct((PP,), jnp.float32),
        mesh=_mesh(),
        compiler_params=pltpu.CompilerParams(needs_layout_passes=False),
        scratch_types=[
            pltpu.VMEM((N,), jnp.float32),
            pltpu.VMEM((N,), jnp.float32),
            pltpu.VMEM((PT,), jnp.int32),
            pltpu.VMEM((PT,), jnp.int32),
            pltpu.VMEM((PT,), jnp.float32),
        ],
    )
    def body(h0_hbm, h1_hbm, srcf_hbm, dstf_hbm, out_hbm,
             c0, c1, si, di, ob):
        wid = lax.axis_index("s") * NC + lax.axis_index("c")
        base = wid * PT
        pltpu.sync_copy(h0_hbm, c0)
        pltpu.sync_copy(h1_hbm, c1)
        pltpu.sync_copy(srcf_hbm.at[pl.ds(base, PT)], si)
        pltpu.sync_copy(dstf_hbm.at[pl.ds(base, PT)], di)

        def step(j, carry):
            sv = si[pl.ds(j * 16, 16)]
            dv = di[pl.ds(j * 16, 16)]
            a0 = plsc.load_gather(c0, [sv])
            a1 = plsc.load_gather(c1, [sv])
            b0 = plsc.load_gather(c0, [dv])
            b1 = plsc.load_gather(c1, [dv])
            d0 = b0 - a0
            d1 = b1 - a1
            ob[pl.ds(j * 16, 16)] = d0 * d0 + d1 * d1
            return carry

        lax.fori_loop(0, DLOOP, step, 0)
        pltpu.sync_copy(ob, out_hbm.at[pl.ds(base, PT)])

    return body


def _sc_dist(h0, h1, srcf, dstf):
    return _sc_dist_kernel()(h0, h1, srcf, dstf)


def _lin_body(h_ref, w_ref, o_ref):
    o_ref[...] = jnp.dot(h_ref[...], w_ref[...],
                         preferred_element_type=jnp.float32)


def _tc_linear(h, w):
    return pl.pallas_call(
        _lin_body,
        out_shape=jax.ShapeDtypeStruct((N, D), jnp.float32),
    )(h, w)


BE = 4000
NEB = E // BE


def _pass1_body(g_ref, a_ref, w1b_ref, b1_ref, sum_ref, sq_ref):
    i = pl.program_id(0)
    ab = a_ref[...].astype(jnp.bfloat16).astype(jnp.float32)
    wb = w1b_ref[...].astype(jnp.bfloat16).astype(jnp.float32)
    u = jnp.maximum((g_ref[...] + ab * wb) + b1_ref[...], 0.0)
    s = jnp.sum(u, axis=0, keepdims=True)
    q = jnp.sum(u * u, axis=0, keepdims=True)

    @pl.when(i == 0)
    def _():
        sum_ref[...] = s
        sq_ref[...] = q

    @pl.when(i > 0)
    def _():
        sum_ref[...] += s
        sq_ref[...] += q


def _tc_pass1(G, attr, w1b, b1):
    return pl.pallas_call(
        _pass1_body,
        grid=(NEB,),
        in_specs=[
            pl.BlockSpec((BE, D), lambda i: (i, 0)),
            pl.BlockSpec((BE, 1), lambda i: (i, 0)),
            pl.BlockSpec((1, D), lambda i: (0, 0)),
            pl.BlockSpec((1, D), lambda i: (0, 0)),
        ],
        out_specs=[
            pl.BlockSpec((1, D), lambda i: (0, 0)),
            pl.BlockSpec((1, D), lambda i: (0, 0)),
        ],
        out_shape=[
            jax.ShapeDtypeStruct((1, D), jnp.float32),
            jax.ShapeDtypeStruct((1, D), jnp.float32),
        ],
    )(G, attr, w1b, b1)


def _pass2_body(g_ref, a_ref, w1b_ref, b1_ref, mu_ref, var_ref, g1_ref,
                be1_ref, w2_ref, b2_ref, v_ref, sq_ref):
    i = pl.program_id(0)
    ab = a_ref[...].astype(jnp.bfloat16).astype(jnp.float32)
    wb = w1b_ref[...].astype(jnp.bfloat16).astype(jnp.float32)
    u = jnp.maximum((g_ref[...] + ab * wb) + b1_ref[...], 0.0)
    uh = ((u - mu_ref[...]) / jnp.sqrt(var_ref[...] + EPS)
          * g1_ref[...] + be1_ref[...])
    v = jnp.maximum(
        jnp.dot(uh, w2_ref[...], preferred_element_type=jnp.float32)
        + b2_ref[...], 0.0)
    v_ref[...] = v
    q = jnp.sum(v * v, axis=0, keepdims=True)

    @pl.when(i == 0)
    def _():
        sq_ref[...] = q

    @pl.when(i > 0)
    def _():
        sq_ref[...] += q


def _tc_pass2(G, attr, w1b, b1, mu1, var1, g1, be1, w2, b2):
    return pl.pallas_call(
        _pass2_body,
        grid=(NEB,),
        in_specs=[
            pl.BlockSpec((BE, D), lambda i: (i, 0)),
            pl.BlockSpec((BE, 1), lambda i: (i, 0)),
            pl.BlockSpec((1, D), lambda i: (0, 0)),
            pl.BlockSpec((1, D), lambda i: (0, 0)),
            pl.BlockSpec((1, D), lambda i: (0, 0)),
            pl.BlockSpec((1, D), lambda i: (0, 0)),
            pl.BlockSpec((1, D), lambda i: (0, 0)),
            pl.BlockSpec((1, D), lambda i: (0, 0)),
            pl.BlockSpec((D, D), lambda i: (0, 0)),
            pl.BlockSpec((1, D), lambda i: (0, 0)),
        ],
        out_specs=[
            pl.BlockSpec((BE, D), lambda i: (i, 0)),
            pl.BlockSpec((1, D), lambda i: (0, 0)),
        ],
        out_shape=[
            jax.ShapeDtypeStruct((E, D), jnp.float32),
            jax.ShapeDtypeStruct((1, D), jnp.float32),
        ],
    )(G, attr, w1b, b1, mu1, var1, g1, be1, w2, b2)


def _node_body(h_ref, s_ref, cnt_ref, q2_ref, g2_ref, be2_ref,
               wa_ref, wb_ref, b1_ref, g1_ref, bb1_ref,
               w2_ref, b2_ref, gg2_ref, bb2_ref, o_ref):
    sv = s_ref[0] + s_ref[1]
    cnt = cnt_ref[0, :, 0:1] + cnt_ref[1, :, 0:1]
    mu2 = jnp.sum(sv, axis=0, keepdims=True) / E
    var2 = q2_ref[...] / E - mu2 * mu2
    gamma = g2_ref[...] / jnp.sqrt(var2 + EPS)
    delta = be2_ref[...] - mu2 * gamma
    agg = (sv * gamma + cnt * delta) / jnp.maximum(cnt, 1.0)

    z = jnp.dot(h_ref[...], wa_ref[...],
                 preferred_element_type=jnp.float32)
    z += jnp.dot(agg, wb_ref[...],
                 preferred_element_type=jnp.float32)
    z = jnp.maximum(z + b1_ref[...], 0.0)
    mu = jnp.mean(z, axis=0, keepdims=True)
    var = jnp.mean((z - mu) ** 2, axis=0, keepdims=True)
    z = (z - mu) / jnp.sqrt(var + EPS) * g1_ref[...] + bb1_ref[...]

    z2 = jnp.maximum(
        jnp.dot(z, w2_ref[...],
                 preferred_element_type=jnp.float32)
        + b2_ref[...], 0.0)
    mu = jnp.mean(z2, axis=0, keepdims=True)
    var = jnp.mean((z2 - mu) ** 2, axis=0, keepdims=True)
    o_ref[...] = (z2 - mu) / jnp.sqrt(var + EPS) * gg2_ref[...] + bb2_ref[...]


def _tc_node(h, s_part, cnt_part, q2, g2, be2, wa, wb, b1, g1, bb1,
             w2, b2, gg2, bb2):
    return pl.pallas_call(
        _node_body,
        out_shape=jax.ShapeDtypeStruct((N, D), jnp.float32),
    )(h, s_part, cnt_part, q2, g2, be2, wa, wb, b1, g1, bb1, w2, b2, gg2, bb2)


def _row(v):
    return v.reshape(1, -1).astype(jnp.float32)


def kernel(x, edge_index, edge_attr, idx, params):
    src = edge_index[1]
    attr = edge_attr.astype(jnp.float32)

    h = x
    cnt_part = None
    for lp in params:
        m1, m2 = lp["msg"]
        n1, n2 = lp["node"]
        w1a = m1["W"][:D]
        w1b = _row(m1["W"][D])
        b1 = _row(m1["b"])

        A = _tc_linear(h, w1a)
        G = _sc_gather(A, src)
        s1, q1 = _tc_pass1(G, attr, w1b, b1)
        mu1 = s1 / E
        var1 = q1 / E - mu1 * mu1
        v, q2 = _tc_pass2(G, attr, w1b, b1, mu1, var1, _row(m1["g"]),
                          _row(m1["be"]), m2["W"], _row(m2["b"]))
        if cnt_part is None:
            s_part, cnt_part = _sc_scatter(v, src, True)
        else:
            s_part = _sc_scatter(v, src, False)

        nout = n2["W"].shape[1]
        w2p = jnp.zeros((D, D), jnp.float32).at[:, :nout].set(n2["W"])
        b2p = jnp.zeros((1, D), jnp.float32).at[:, :nout].set(_row(n2["b"]))
        g2p = jnp.ones((1, D), jnp.float32).at[:, :nout].set(_row(n2["g"]))
        be2p = jnp.zeros((1, D), jnp.float32).at[:, :nout].set(_row(n2["be"]))

        h = _tc_node(h, s_part, cnt_part, q2, _row(m2["g"]), _row(m2["be"]),
                     n1["W"][:D], n1["W"][D:], _row(n1["b"]), _row(n1["g"]),
                     _row(n1["be"]), w2p, b2p, g2p, be2p)

    srcf = jnp.concatenate(
        [idx.reshape(-1), jnp.zeros((PP - N * K,), jnp.int32)])
    dstf = jnp.concatenate(
        [jnp.repeat(jnp.arange(N, dtype=jnp.int32), K),
         jnp.zeros((PP - N * K,), jnp.int32)])
    dists = _sc_dist(h[:, 0], h[:, 1], srcf, dstf)
    return dists[:N * K].reshape(-1, 1)


# speedup vs baseline: 4.1818x; 1.0768x over previous
import functools

import jax
import jax.numpy as jnp
from jax import lax
from jax.experimental import pallas as pl
from jax.experimental.pallas import tpu as pltpu
from jax.experimental.pallas import tpu_sc as plsc

N = 10000
K = 17
D = 128
E = N * (K - 1)
EPS = 1e-5

NC, NS = 2, 16
NW = NC * NS
CH = 128
NCHUNK = E // CH
CPW = -(-NCHUNK // NW)
RPT = 632

PT = 5328
PP = PT * NW
DLOOP = PT // 16

def _mesh():
    return plsc.VectorSubcoreMesh(
        core_axis_name="c", subcore_axis_name="s",
        num_cores=NC, num_subcores=NS)


@functools.cache
def _sc_gather_kernel():
    @functools.partial(
        pl.kernel,
        out_type=jax.ShapeDtypeStruct((E, D), jnp.float32),
        mesh=_mesh(),
        scratch_types=[
            pltpu.VMEM((CH,), jnp.int32),
            pltpu.VMEM((CH,), jnp.int32),
            pltpu.VMEM((CH,), jnp.int32),
            pltpu.VMEM((CH, D), jnp.float32),
            pltpu.VMEM((CH, D), jnp.float32),
            pltpu.VMEM((CH, D), jnp.float32),
            pltpu.SemaphoreType.DMA,
            pltpu.SemaphoreType.DMA,
            pltpu.SemaphoreType.DMA,
        ],
    )
    def body(table_hbm, src_hbm, out_hbm, idx0, idx1, idx2,
             rows0, rows1, rows2, sem0, sem1, sem2):
        idxs = (idx0, idx1, idx2)
        rows = (rows0, rows1, rows2)
        sems = (sem0, sem1, sem2)
        wid = lax.axis_index("s") * NC + lax.axis_index("c")
        for i in range(CPW + 2):
            b = i % 3
            if i < CPW:
                cid = wid + i * NW

                @pl.when(cid < NCHUNK)
                def _():
                    base = cid * CH
                    pltpu.sync_copy(src_hbm.at[pl.ds(base, CH)], idxs[b])
                    pltpu.async_copy(table_hbm.at[idxs[b]], rows[b], sems[b])
            if i > 1:
                pb = (i - 2) % 3
                pcid = wid + (i - 2) * NW

                @pl.when(pcid < NCHUNK)
                def _():
                    pltpu.make_async_copy(
                        table_hbm.at[pl.ds(0, CH)], rows[pb],
                        sems[pb]).wait()
                    pltpu.sync_copy(rows[pb],
                                    out_hbm.at[pl.ds(pcid * CH, CH)])

    return body


def _sc_gather(table, src):
    return _sc_gather_kernel()(table, src)


@functools.cache
def _sc_scatter_kernel(count_only):
    @functools.partial(
        pl.kernel,
        out_type=jax.ShapeDtypeStruct((NC, N, D), jnp.float32),
        mesh=_mesh(),
        scratch_types=[
            pltpu.VMEM((CH,), jnp.int32),
            pltpu.VMEM((CH,), jnp.int32),
            pltpu.VMEM((CH, D), jnp.float32),
            pltpu.VMEM((CH, D), jnp.float32),
            pltpu.SemaphoreType.DMA,
            pltpu.SemaphoreType.DMA,
            pltpu.VMEM_SHARED((N, D), jnp.float32),
        ],
    )
    def body(*args):
        if count_only:
            v_hbm = None
            src_hbm, s_out, idx0, idx1, rows0, rows1, sem0, sem1, s_sh = args
        else:
            (v_hbm, src_hbm, s_out, idx0, idx1, rows0, rows1, sem0, sem1,
             s_sh) = args
        idxs = (idx0, idx1)
        rows = (rows0, rows1)
        sems = (sem0, sem1)
        idx_v, rows_v = idx0, rows0
        c = lax.axis_index("c")
        s = lax.axis_index("s")
        wid = s * NC + c
        row0 = jnp.minimum(s * RPT, N - RPT)
        pieces = []
        off = 0
        while off < RPT:
            pieces.append((off, min(CH, RPT - off)))
            off += CH
        zeros16 = jnp.zeros((16,), jnp.float32)
        ones16 = jnp.ones((16,), jnp.float32)

        def fill(val):
            def frow(r, carry):
                for j in range(D // 16):
                    rows_v[r, pl.ds(j * 16, 16)] = val
                return carry
            lax.fori_loop(0, CH, frow, 0)

        def zero_accum():
            fill(zeros16)
            for off, sz in pieces:
                pltpu.sync_copy(rows_v.at[pl.ds(0, sz)],
                                s_sh.at[pl.ds(row0 + off, sz)])

        def accumulate(load_rows):
            zero_accum()
            if not load_rows:
                fill(ones16)
            plsc.subcore_barrier()
            for i in range(CPW + 1):
                b = i & 1
                if i < CPW:
                    cid = wid + i * NW

                    @pl.when(cid < NCHUNK)
                    def _():
                        base = cid * CH
                        pltpu.sync_copy(src_hbm.at[pl.ds(base, CH)], idxs[b])
                        if load_rows:
                            pltpu.async_copy(v_hbm.at[pl.ds(base, CH)],
                                             rows[b], sems[b])
                if i > 0:
                    pb = 1 - b
                    pcid = wid + (i - 1) * NW

                    @pl.when(pcid < NCHUNK)
                    def _():
                        if load_rows:
                            pltpu.make_async_copy(
                                v_hbm.at[pl.ds(0, CH)], rows[pb],
                                sems[pb]).wait()
                            pltpu.sync_copy(rows[pb], s_sh.at[idxs[pb]],
                                            add=True)
                        else:
                            pltpu.sync_copy(rows_v, s_sh.at[idxs[pb]],
                                            add=True)
            plsc.subcore_barrier()

        def drain(out_ref):
            for off, sz in pieces:
                pltpu.sync_copy(s_sh.at[pl.ds(row0 + off, sz)],
                                rows_v.at[pl.ds(0, sz)])
                pltpu.sync_copy(rows_v.at[pl.ds(0, sz)],
                                out_ref.at[c, pl.ds(row0 + off, sz)])

        accumulate(not count_only)
        drain(s_out)

    return body


def _sc_scatter(v, src):
    return _sc_scatter_kernel(False)(v, src)


def _sc_cnt(src):
    return _sc_scatter_kernel(True)(src)


@functools.cache
def _sc_dist_kernel():
    @functools.partial(
        pl.kernel,
        out_type=jax.ShapeDtypeStruct((PP,), jnp.float32),
        mesh=_mesh(),
        compiler_params=pltpu.CompilerParams(needs_layout_passes=False),
        scratch_types=[
            pltpu.VMEM((N,), jnp.float32),
            pltpu.VMEM((N,), jnp.float32),
            pltpu.VMEM((PT,), jnp.int32),
            pltpu.VMEM((PT,), jnp.int32),
            pltpu.VMEM((PT,), jnp.float32),
        ],
    )
    def body(h0_hbm, h1_hbm, srcf_hbm, dstf_hbm, out_hbm,
             c0, c1, si, di, ob):
        wid = lax.axis_index("s") * NC + lax.axis_index("c")
        base = wid * PT
        pltpu.sync_copy(h0_hbm, c0)
        pltpu.sync_copy(h1_hbm, c1)
        pltpu.sync_copy(srcf_hbm.at[pl.ds(base, PT)], si)
        pltpu.sync_copy(dstf_hbm.at[pl.ds(base, PT)], di)

        def step(j, carry):
            sv = si[pl.ds(j * 16, 16)]
            dv = di[pl.ds(j * 16, 16)]
            a0 = plsc.load_gather(c0, [sv])
            a1 = plsc.load_gather(c1, [sv])
            b0 = plsc.load_gather(c0, [dv])
            b1 = plsc.load_gather(c1, [dv])
            d0 = b0 - a0
            d1 = b1 - a1
            ob[pl.ds(j * 16, 16)] = d0 * d0 + d1 * d1
            return carry

        lax.fori_loop(0, DLOOP, step, 0)
        pltpu.sync_copy(ob, out_hbm.at[pl.ds(base, PT)])

    return body


def _sc_dist(h0, h1, srcf, dstf):
    return _sc_dist_kernel()(h0, h1, srcf, dstf)


def _lin_body(h_ref, w_ref, o_ref):
    o_ref[...] = jnp.dot(h_ref[...], w_ref[...],
                         preferred_element_type=jnp.float32)


def _tc_linear(h, w):
    return pl.pallas_call(
        _lin_body,
        out_shape=jax.ShapeDtypeStruct((N, D), jnp.float32),
    )(h, w)


BE = 4000
NEB = E // BE


def _pass1_body(g_ref, a_ref, w1b_ref, b1_ref, sum_ref, sq_ref):
    i = pl.program_id(0)
    ab = a_ref[...].astype(jnp.bfloat16).astype(jnp.float32)
    wb = w1b_ref[...].astype(jnp.bfloat16).astype(jnp.float32)
    u = jnp.maximum((g_ref[...] + ab * wb) + b1_ref[...], 0.0)
    s = jnp.sum(u, axis=0, keepdims=True)
    q = jnp.sum(u * u, axis=0, keepdims=True)

    @pl.when(i == 0)
    def _():
        sum_ref[...] = s
        sq_ref[...] = q

    @pl.when(i > 0)
    def _():
        sum_ref[...] += s
        sq_ref[...] += q


def _tc_pass1(G, attr, w1b, b1):
    return pl.pallas_call(
        _pass1_body,
        grid=(NEB,),
        in_specs=[
            pl.BlockSpec((BE, D), lambda i: (i, 0)),
            pl.BlockSpec((BE, 1), lambda i: (i, 0)),
            pl.BlockSpec((1, D), lambda i: (0, 0)),
            pl.BlockSpec((1, D), lambda i: (0, 0)),
        ],
        out_specs=[
            pl.BlockSpec((1, D), lambda i: (0, 0)),
            pl.BlockSpec((1, D), lambda i: (0, 0)),
        ],
        out_shape=[
            jax.ShapeDtypeStruct((1, D), jnp.float32),
            jax.ShapeDtypeStruct((1, D), jnp.float32),
        ],
    )(G, attr, w1b, b1)


def _pass2_body(g_ref, a_ref, w1b_ref, b1_ref, mu_ref, var_ref, g1_ref,
                be1_ref, w2_ref, b2_ref, v_ref, sq_ref):
    i = pl.program_id(0)
    ab = a_ref[...].astype(jnp.bfloat16).astype(jnp.float32)
    wb = w1b_ref[...].astype(jnp.bfloat16).astype(jnp.float32)
    u = jnp.maximum((g_ref[...] + ab * wb) + b1_ref[...], 0.0)
    uh = ((u - mu_ref[...]) / jnp.sqrt(var_ref[...] + EPS)
          * g1_ref[...] + be1_ref[...])
    v = jnp.maximum(
        jnp.dot(uh, w2_ref[...], preferred_element_type=jnp.float32)
        + b2_ref[...], 0.0)
    v_ref[...] = v
    q = jnp.sum(v * v, axis=0, keepdims=True)

    @pl.when(i == 0)
    def _():
        sq_ref[...] = q

    @pl.when(i > 0)
    def _():
        sq_ref[...] += q


def _tc_pass2(G, attr, w1b, b1, mu1, var1, g1, be1, w2, b2):
    return pl.pallas_call(
        _pass2_body,
        grid=(NEB,),
        in_specs=[
            pl.BlockSpec((BE, D), lambda i: (i, 0)),
            pl.BlockSpec((BE, 1), lambda i: (i, 0)),
            pl.BlockSpec((1, D), lambda i: (0, 0)),
            pl.BlockSpec((1, D), lambda i: (0, 0)),
            pl.BlockSpec((1, D), lambda i: (0, 0)),
            pl.BlockSpec((1, D), lambda i: (0, 0)),
            pl.BlockSpec((1, D), lambda i: (0, 0)),
            pl.BlockSpec((1, D), lambda i: (0, 0)),
            pl.BlockSpec((D, D), lambda i: (0, 0)),
            pl.BlockSpec((1, D), lambda i: (0, 0)),
        ],
        out_specs=[
            pl.BlockSpec((BE, D), lambda i: (i, 0)),
            pl.BlockSpec((1, D), lambda i: (0, 0)),
        ],
        out_shape=[
            jax.ShapeDtypeStruct((E, D), jnp.float32),
            jax.ShapeDtypeStruct((1, D), jnp.float32),
        ],
    )(G, attr, w1b, b1, mu1, var1, g1, be1, w2, b2)


def _node_body(h_ref, s_ref, cnt_ref, q2_ref, g2_ref, be2_ref,
               wa_ref, wb_ref, b1_ref, g1_ref, bb1_ref,
               w2_ref, b2_ref, gg2_ref, bb2_ref, o_ref):
    sv = s_ref[0] + s_ref[1]
    cnt = cnt_ref[0, :, 0:1] + cnt_ref[1, :, 0:1]
    mu2 = jnp.sum(sv, axis=0, keepdims=True) / E
    var2 = q2_ref[...] / E - mu2 * mu2
    gamma = g2_ref[...] / jnp.sqrt(var2 + EPS)
    delta = be2_ref[...] - mu2 * gamma
    agg = (sv * gamma + cnt * delta) / jnp.maximum(cnt, 1.0)

    z = jnp.dot(h_ref[...], wa_ref[...],
                 preferred_element_type=jnp.float32)
    z += jnp.dot(agg, wb_ref[...],
                 preferred_element_type=jnp.float32)
    z = jnp.maximum(z + b1_ref[...], 0.0)
    mu = jnp.mean(z, axis=0, keepdims=True)
    var = jnp.mean((z - mu) ** 2, axis=0, keepdims=True)
    z = (z - mu) / jnp.sqrt(var + EPS) * g1_ref[...] + bb1_ref[...]

    z2 = jnp.maximum(
        jnp.dot(z, w2_ref[...],
                 preferred_element_type=jnp.float32)
        + b2_ref[...], 0.0)
    mu = jnp.mean(z2, axis=0, keepdims=True)
    var = jnp.mean((z2 - mu) ** 2, axis=0, keepdims=True)
    o_ref[...] = (z2 - mu) / jnp.sqrt(var + EPS) * gg2_ref[...] + bb2_ref[...]


def _tc_node(h, s_part, cnt_part, q2, g2, be2, wa, wb, b1, g1, bb1,
             w2, b2, gg2, bb2):
    return pl.pallas_call(
        _node_body,
        out_shape=jax.ShapeDtypeStruct((N, D), jnp.float32),
    )(h, s_part, cnt_part, q2, g2, be2, wa, wb, b1, g1, bb1, w2, b2, gg2, bb2)


def _row(v):
    return v.reshape(1, -1).astype(jnp.float32)


def kernel(x, edge_index, edge_attr, idx, params):
    src = edge_index[1]
    attr = edge_attr.astype(jnp.float32)

    h = x
    cnt_part = _sc_cnt(src)
    for lp in params:
        m1, m2 = lp["msg"]
        n1, n2 = lp["node"]
        w1a = m1["W"][:D]
        w1b = _row(m1["W"][D])
        b1 = _row(m1["b"])

        A = _tc_linear(h, w1a)
        G = _sc_gather(A, src)
        s1, q1 = _tc_pass1(G, attr, w1b, b1)
        mu1 = s1 / E
        var1 = q1 / E - mu1 * mu1
        v, q2 = _tc_pass2(G, attr, w1b, b1, mu1, var1, _row(m1["g"]),
                          _row(m1["be"]), m2["W"], _row(m2["b"]))
        s_part = _sc_scatter(v, src)

        nout = n2["W"].shape[1]
        w2p = jnp.zeros((D, D), jnp.float32).at[:, :nout].set(n2["W"])
        b2p = jnp.zeros((1, D), jnp.float32).at[:, :nout].set(_row(n2["b"]))
        g2p = jnp.ones((1, D), jnp.float32).at[:, :nout].set(_row(n2["g"]))
        be2p = jnp.zeros((1, D), jnp.float32).at[:, :nout].set(_row(n2["be"]))

        h = _tc_node(h, s_part, cnt_part, q2, _row(m2["g"]), _row(m2["be"]),
                     n1["W"][:D], n1["W"][D:], _row(n1["b"]), _row(n1["g"]),
                     _row(n1["be"]), w2p, b2p, g2p, be2p)

    srcf = jnp.concatenate(
        [idx.reshape(-1), jnp.zeros((PP - N * K,), jnp.int32)])
    dstf = jnp.concatenate(
        [jnp.repeat(jnp.arange(N, dtype=jnp.int32), K),
         jnp.zeros((PP - N * K,), jnp.int32)])
    dists = _sc_dist(h[:, 0], h[:, 1], srcf, dstf)
    return dists[:N * K].reshape(-1, 1)
